# Initial kernel scaffold; baseline (speedup 1.0000x reference)
import jax
import jax.numpy as jnp
from jax.experimental import pallas as pl


def kernel(x, edge_index, W0, b0, g0, be0, W1, b1, g1, be1, W2, b2, g2, be2):
    raise NotImplementedError("write your pallas kernel here")


# speedup vs baseline: 25.7595x; 25.7595x over previous
import functools

import jax
import jax.numpy as jnp
from jax import lax
from jax.experimental import pallas as pl
from jax.experimental.pallas import tpu as pltpu
from jax.experimental.pallas import tpu_sc as plsc

N = 10000
D = 128
NC = 2
NS = 16
NW = NC * NS
CHUNK = 128
GRP = 8

DEG_ROWS = 10240
DEG_PER_TILE = DEG_ROWS // NS

ACC_ROWS = 10112
ACC_PER_TILE = ACC_ROWS // NS
EPS = 1e-5

_MESH = plsc.VectorSubcoreMesh(core_axis_name="c", subcore_axis_name="s")


def _deg_body(nchunks, dst_hbm, zeros_hbm, ones_hbm, deg_out, idx_v, ones_v,
              stage_v, acc_sh):
  c = lax.axis_index("c")
  s = lax.axis_index("s")
  wid = c * NS + s
  pltpu.sync_copy(dst_hbm.at[wid], idx_v)
  pltpu.sync_copy(ones_hbm, ones_v)
  pltpu.sync_copy(zeros_hbm, stage_v)
  pltpu.sync_copy(stage_v, acc_sh.at[pl.ds(s * DEG_PER_TILE, DEG_PER_TILE)])
  plsc.subcore_barrier()
  for j in range(nchunks):
    pltpu.sync_copy(ones_v, acc_sh.at[idx_v.at[j]], add=True)
  plsc.subcore_barrier()
  pltpu.sync_copy(acc_sh.at[pl.ds(s * DEG_PER_TILE, DEG_PER_TILE)], stage_v)
  pltpu.sync_copy(stage_v,
                  deg_out.at[c, pl.ds(s * DEG_PER_TILE, DEG_PER_TILE)])


def _spmm_body(nchunks, xs_hbm, src_hbm, dst_hbm, zeros_hbm, pout_hbm,
               sidx_v, dstidx_v, rows_v, semi, semg0, semg1, acc_sh):
  c = lax.axis_index("c")
  s = lax.axis_index("s")
  wid = c * NS + s
  ngrp = nchunks // GRP
  pltpu.sync_copy(dst_hbm.at[wid], dstidx_v)
  pltpu.sync_copy(src_hbm.at[wid, pl.ds(0, GRP)], sidx_v.at[0])

  pltpu.sync_copy(zeros_hbm, rows_v.at[0])
  base = s * ACC_PER_TILE
  for k in range(ACC_PER_TILE // CHUNK):
    pltpu.sync_copy(rows_v.at[0], acc_sh.at[pl.ds(base + k * CHUNK, CHUNK)])
  rem = ACC_PER_TILE % CHUNK
  if rem:
    pltpu.sync_copy(
        rows_v.at[0, pl.ds(0, rem)],
        acc_sh.at[pl.ds(base + ACC_PER_TILE - rem, rem)])
  plsc.subcore_barrier()

  semg = (semg0, semg1)
  icp = None
  cp = pltpu.async_copy(xs_hbm.at[sidx_v.at[0, 0]], rows_v.at[0], semg[0])
  for j in range(nchunks):
    g, i = divmod(j, GRP)
    if i == 0 and g + 1 < ngrp:
      icp = pltpu.async_copy(src_hbm.at[wid, pl.ds((g + 1) * GRP, GRP)],
                             sidx_v.at[(g + 1) % 2], semi)
    if j + 1 < nchunks:
      g1, i1 = divmod(j + 1, GRP)
      if i1 == 0:
        icp.wait()
      nxt = pltpu.async_copy(xs_hbm.at[sidx_v.at[g1 % 2, i1]],
                             rows_v.at[(j + 1) % 2], semg[(j + 1) % 2])
    cp.wait()
    pltpu.sync_copy(rows_v.at[j % 2], acc_sh.at[dstidx_v.at[j]], add=True)
    if j + 1 < nchunks:
      cp = nxt
  plsc.subcore_barrier()

  for k in range(ACC_PER_TILE // CHUNK):
    r0 = base + k * CHUNK
    pltpu.sync_copy(acc_sh.at[pl.ds(r0, CHUNK)], rows_v.at[0])
    pltpu.sync_copy(rows_v.at[0], pout_hbm.at[c, pl.ds(r0, CHUNK)])
  if rem:
    r0 = base + ACC_PER_TILE - rem
    pltpu.sync_copy(acc_sh.at[pl.ds(r0, rem)], rows_v.at[0, pl.ds(0, rem)])
    pltpu.sync_copy(rows_v.at[0, pl.ds(0, rem)],
                    pout_hbm.at[c, pl.ds(r0, rem)])


def _make_deg_kernel(nchunks):
  return pl.kernel(
      functools.partial(_deg_body, nchunks),
      out_type=jax.ShapeDtypeStruct((NC, DEG_ROWS), jnp.float32),
      mesh=_MESH,
      scratch_types=[
          pltpu.VMEM((nchunks, CHUNK), jnp.int32),
          pltpu.VMEM((CHUNK,), jnp.float32),
          pltpu.VMEM((DEG_PER_TILE,), jnp.float32),
          pltpu.VMEM_SHARED((DEG_ROWS,), jnp.float32),
      ],
  )


def _make_spmm_kernel(nchunks):
  return pl.kernel(
      functools.partial(_spmm_body, nchunks),
      out_type=jax.ShapeDtypeStruct((NC, ACC_ROWS, D), jnp.float32),
      mesh=_MESH,
      scratch_types=[
          pltpu.VMEM((2, GRP, CHUNK), jnp.int32),
          pltpu.VMEM((nchunks, CHUNK), jnp.int32),
          pltpu.VMEM((2, CHUNK, D), jnp.float32),
          pltpu.SemaphoreType.DMA,
          pltpu.SemaphoreType.DMA,
          pltpu.SemaphoreType.DMA,
          pltpu.VMEM_SHARED((ACC_ROWS, D), jnp.float32),
      ],
  )


def _bn_xw(h, g, be, w, dinv):
  mean = jnp.mean(h, axis=0, keepdims=True)
  dlt = h - mean
  var = jnp.mean(dlt * dlt, axis=0, keepdims=True)
  hb = g * dlt * lax.rsqrt(var + EPS) + be
  return dinv * jnp.dot(hb, w, preferred_element_type=jnp.float32)


def _tc_front_body(x_ref, dinv_ref, w_ref, g_ref, be_ref, xs_ref):
  xs_ref[...] = _bn_xw(x_ref[...], g_ref[...], be_ref[...], w_ref[...],
                       dinv_ref[...])


def _tc_mid_body(p_ref, xs_ref, dinv_ref, b_ref, w_ref, g_ref, be_ref,
                 h_ref, xsn_ref):
  dinv = dinv_ref[...]
  ssum = p_ref[0, :N] + p_ref[1, :N] + xs_ref[...]
  h = jnp.maximum(dinv * ssum + b_ref[...], 0.0)
  h_ref[...] = h
  xsn_ref[...] = _bn_xw(h, g_ref[...], be_ref[...], w_ref[...], dinv)


def _tc_last_body(p_ref, xs_ref, dinv_ref, b_ref, h_ref):
  ssum = p_ref[0, :N] + p_ref[1, :N] + xs_ref[...]
  h_ref[...] = jnp.maximum(dinv_ref[...] * ssum + b_ref[...], 0.0)


_tc_front = pl.pallas_call(
    _tc_front_body,
    out_shape=jax.ShapeDtypeStruct((N, D), jnp.float32),
)

_tc_mid = pl.pallas_call(
    _tc_mid_body,
    out_shape=(
        jax.ShapeDtypeStruct((N, D), jnp.float32),
        jax.ShapeDtypeStruct((N, D), jnp.float32),
    ),
)

_tc_last = pl.pallas_call(
    _tc_last_body,
    out_shape=jax.ShapeDtypeStruct((N, D), jnp.float32),
)


@jax.jit
def _run(x, edge_index, W0, b0, g0, be0, W1, b1, g1, be1, W2, b2, g2, be2):
  e = edge_index.shape[1]

  ncd = -(-e // (NW * CHUNK))
  padd = NW * ncd * CHUNK - e
  iotad = jnp.arange(padd, dtype=jnp.int32)
  dst_deg = jnp.concatenate(
      [edge_index[1], N + iotad % (DEG_ROWS - N)]).reshape(NW, ncd, CHUNK)

  ncs = -(-e // (NW * CHUNK * GRP)) * GRP
  pads = NW * ncs * CHUNK - e
  iotas = jnp.arange(pads, dtype=jnp.int32)
  src_p = jnp.concatenate(
      [edge_index[0], (iotas * 97) % N]).reshape(NW, ncs, CHUNK)
  dst_p = jnp.concatenate(
      [edge_index[1], N + iotas % (ACC_ROWS - N)]).reshape(NW, ncs, CHUNK)

  zeros_deg = jnp.zeros((DEG_PER_TILE,), jnp.float32)
  zeros_blk = jnp.zeros((CHUNK, D), jnp.float32)
  ones_row = jnp.ones((CHUNK,), jnp.float32)

  deg_p = _make_deg_kernel(ncd)(dst_deg, zeros_deg, ones_row)
  dinv = lax.rsqrt(deg_p[0, :N] + deg_p[1, :N] + 1.0).reshape(N, 1)

  spmm = _make_spmm_kernel(ncs)
  g0r, be0r, b0r = g0.reshape(1, D), be0.reshape(1, D), b0.reshape(1, D)
  g1r, be1r, b1r = g1.reshape(1, D), be1.reshape(1, D), b1.reshape(1, D)
  g2r, be2r, b2r = g2.reshape(1, D), be2.reshape(1, D), b2.reshape(1, D)

  xs0 = _tc_front(x, dinv, W0, g0r, be0r)
  p0 = spmm(xs0, src_p, dst_p, zeros_blk)
  h1, xs1 = _tc_mid(p0, xs0, dinv, b0r, W1, g1r, be1r)
  p1 = spmm(xs1, src_p, dst_p, zeros_blk)
  h2, xs2 = _tc_mid(p1, xs1, dinv, b1r, W2, g2r, be2r)
  p2 = spmm(xs2, src_p, dst_p, zeros_blk)
  h3 = _tc_last(p2, xs2, dinv, b2r)
  return jnp.concatenate([h1, h2, h3], axis=-1)


def kernel(x, edge_index, W0, b0, g0, be0, W1, b1, g1, be1, W2, b2, g2, be2):
  return _run(x, edge_index, W0, b0, g0, be0, W1, b1, g1, be1, W2, b2, g2,
              be2)


# speedup vs baseline: 26.3280x; 1.0221x over previous
import functools

import jax
import jax.numpy as jnp
from jax import lax
from jax.experimental import pallas as pl
from jax.experimental.pallas import tpu as pltpu
from jax.experimental.pallas import tpu_sc as plsc

N = 10000
D = 128
NC = 2
NS = 16
NW = NC * NS
CHUNK = 128
GRP = 8

ACC_ROWS = 10112
ACC_PER_TILE = ACC_ROWS // NS
EPS = 1e-5
PAD = -1

_MESH = plsc.VectorSubcoreMesh(core_axis_name="c", subcore_axis_name="s")


def _deg_body(nchunks, dst_hbm, zeros_hbm, ones_hbm, deg_out, idx_v, ones_v,
              stage_v, acc_sh):
  c = lax.axis_index("c")
  s = lax.axis_index("s")
  wid = c * NS + s
  pltpu.sync_copy(dst_hbm.at[wid], idx_v)
  pltpu.sync_copy(ones_hbm, ones_v)
  pltpu.sync_copy(zeros_hbm, stage_v)
  pltpu.sync_copy(stage_v, acc_sh.at[pl.ds(s * ACC_PER_TILE, ACC_PER_TILE)])
  plsc.subcore_barrier()
  for j in range(nchunks):
    pltpu.sync_copy(ones_v, acc_sh.at[plsc.Indices(idx_v.at[j], PAD)],
                    add=True)
  plsc.subcore_barrier()
  pltpu.sync_copy(acc_sh.at[pl.ds(s * ACC_PER_TILE, ACC_PER_TILE)], stage_v)
  pltpu.sync_copy(stage_v, deg_out.at[c, s, 0])


def _spmm_body(nchunks, xs_hbm, src_hbm, dst_hbm, zeros_hbm, pout_hbm,
               sidx_v, dstidx_v, rows_v, semi, semg0, semg1, acc_sh):
  c = lax.axis_index("c")
  s = lax.axis_index("s")
  wid = c * NS + s
  ngrp = nchunks // GRP
  pltpu.sync_copy(dst_hbm.at[wid], dstidx_v)
  pltpu.sync_copy(src_hbm.at[wid, pl.ds(0, GRP)], sidx_v.at[0])

  pltpu.sync_copy(zeros_hbm, rows_v.at[0])
  base = s * ACC_PER_TILE
  for k in range(ACC_PER_TILE // CHUNK):
    pltpu.sync_copy(rows_v.at[0], acc_sh.at[pl.ds(base + k * CHUNK, CHUNK)])
  rem = ACC_PER_TILE % CHUNK
  if rem:
    pltpu.sync_copy(
        rows_v.at[0, pl.ds(0, rem)],
        acc_sh.at[pl.ds(base + ACC_PER_TILE - rem, rem)])
  plsc.subcore_barrier()

  semg = (semg0, semg1)
  icp = None
  cp = pltpu.async_copy(xs_hbm.at[plsc.Indices(sidx_v.at[0, 0], PAD)],
                        rows_v.at[0], semg[0])
  for j in range(nchunks):
    g, i = divmod(j, GRP)
    if i == 0 and g + 1 < ngrp:
      icp = pltpu.async_copy(src_hbm.at[wid, pl.ds((g + 1) * GRP, GRP)],
                             sidx_v.at[(g + 1) % 2], semi)
    if j + 1 < nchunks:
      g1, i1 = divmod(j + 1, GRP)
      if i1 == 0:
        icp.wait()
      nxt = pltpu.async_copy(
          xs_hbm.at[plsc.Indices(sidx_v.at[g1 % 2, i1], PAD)],
          rows_v.at[(j + 1) % 2], semg[(j + 1) % 2])
    cp.wait()
    pltpu.sync_copy(rows_v.at[j % 2],
                    acc_sh.at[plsc.Indices(dstidx_v.at[j], PAD)], add=True)
    if j + 1 < nchunks:
      cp = nxt
  plsc.subcore_barrier()

  for k in range(ACC_PER_TILE // CHUNK):
    r0 = base + k * CHUNK
    pltpu.sync_copy(acc_sh.at[pl.ds(r0, CHUNK)], rows_v.at[0])
    pltpu.sync_copy(rows_v.at[0], pout_hbm.at[c, pl.ds(r0, CHUNK)])
  if rem:
    r0 = base + ACC_PER_TILE - rem
    pltpu.sync_copy(acc_sh.at[pl.ds(r0, rem)], rows_v.at[0, pl.ds(0, rem)])
    pltpu.sync_copy(rows_v.at[0, pl.ds(0, rem)],
                    pout_hbm.at[c, pl.ds(r0, rem)])


def _make_deg_kernel(nchunks):
  return pl.kernel(
      functools.partial(_deg_body, nchunks),
      out_type=jax.ShapeDtypeStruct((NC, NS, 1, ACC_PER_TILE), jnp.float32),
      mesh=_MESH,
      scratch_types=[
          pltpu.VMEM((nchunks, CHUNK), jnp.int32),
          pltpu.VMEM((CHUNK,), jnp.float32),
          pltpu.VMEM((ACC_PER_TILE,), jnp.float32),
          pltpu.VMEM_SHARED((ACC_ROWS,), jnp.float32),
      ],
  )


def _make_spmm_kernel(nchunks):
  return pl.kernel(
      functools.partial(_spmm_body, nchunks),
      out_type=jax.ShapeDtypeStruct((NC, ACC_ROWS, D), jnp.float32),
      mesh=_MESH,
      scratch_types=[
          pltpu.VMEM((2, GRP, CHUNK), jnp.int32),
          pltpu.VMEM((nchunks, CHUNK), jnp.int32),
          pltpu.VMEM((2, CHUNK, D), jnp.float32),
          pltpu.SemaphoreType.DMA,
          pltpu.SemaphoreType.DMA,
          pltpu.SemaphoreType.DMA,
          pltpu.VMEM_SHARED((ACC_ROWS, D), jnp.float32),
      ],
  )


def _bn_xw(h, g, be, w, dinv):
  mean = jnp.mean(h, axis=0, keepdims=True)
  dlt = h - mean
  var = jnp.mean(dlt * dlt, axis=0, keepdims=True)
  hb = g * dlt * lax.rsqrt(var + EPS) + be
  return dinv * jnp.dot(hb, w, preferred_element_type=jnp.float32)


def _tc_front_body(x_ref, dinv_ref, w_ref, g_ref, be_ref, xs_ref):
  xs_ref[...] = _bn_xw(x_ref[...], g_ref[...], be_ref[...], w_ref[...],
                       dinv_ref[...])


def _tc_mid_body(p_ref, xs_ref, dinv_ref, b_ref, w_ref, g_ref, be_ref,
                 h_ref, xsn_ref):
  dinv = dinv_ref[...]
  ssum = p_ref[0, :N] + p_ref[1, :N] + xs_ref[...]
  h = jnp.maximum(dinv * ssum + b_ref[...], 0.0)
  h_ref[...] = h
  xsn_ref[...] = _bn_xw(h, g_ref[...], be_ref[...], w_ref[...], dinv)


def _tc_last_body(p_ref, xs_ref, dinv_ref, b_ref, h1_ref, h2_ref, out_ref):
  ssum = p_ref[0, :N] + p_ref[1, :N] + xs_ref[...]
  h3 = jnp.maximum(dinv_ref[...] * ssum + b_ref[...], 0.0)
  out_ref[...] = jnp.concatenate([h1_ref[...], h2_ref[...], h3], axis=-1)


_tc_front = pl.pallas_call(
    _tc_front_body,
    out_shape=jax.ShapeDtypeStruct((N, D), jnp.float32),
)

_tc_mid = pl.pallas_call(
    _tc_mid_body,
    out_shape=(
        jax.ShapeDtypeStruct((N, D), jnp.float32),
        jax.ShapeDtypeStruct((N, D), jnp.float32),
    ),
)

_tc_last = pl.pallas_call(
    _tc_last_body,
    out_shape=jax.ShapeDtypeStruct((N, 3 * D), jnp.float32),
)


@jax.jit
def _run(x, edge_index, W0, b0, g0, be0, W1, b1, g1, be1, W2, b2, g2, be2):
  e = edge_index.shape[1]
  ncs = -(-e // (NW * CHUNK * GRP)) * GRP

  if e % NW == 0:
    ppt = ncs * CHUNK - e // NW
    padv = jnp.full((NW, ppt), PAD, jnp.int32)
    src_p = jnp.concatenate(
        [edge_index[0].reshape(NW, e // NW), padv], axis=1
    ).reshape(NW, ncs, CHUNK)
    dst_p = jnp.concatenate(
        [edge_index[1].reshape(NW, e // NW), padv], axis=1
    ).reshape(NW, ncs, CHUNK)
  else:
    pads = NW * ncs * CHUNK - e
    padv = jnp.full((pads,), PAD, jnp.int32)
    src_p = jnp.concatenate([edge_index[0], padv]).reshape(NW, ncs, CHUNK)
    dst_p = jnp.concatenate([edge_index[1], padv]).reshape(NW, ncs, CHUNK)

  zeros_deg = jnp.zeros((ACC_PER_TILE,), jnp.float32)
  zeros_blk = jnp.zeros((CHUNK, D), jnp.float32)
  ones_row = jnp.ones((CHUNK,), jnp.float32)

  deg_p = _make_deg_kernel(ncs)(dst_p, zeros_deg, ones_row)
  deg_p = deg_p.reshape(NC, ACC_ROWS)
  dinv = lax.rsqrt(deg_p[0, :N] + deg_p[1, :N] + 1.0).reshape(N, 1)

  spmm = _make_spmm_kernel(ncs)
  g0r, be0r, b0r = g0.reshape(1, D), be0.reshape(1, D), b0.reshape(1, D)
  g1r, be1r, b1r = g1.reshape(1, D), be1.reshape(1, D), b1.reshape(1, D)
  g2r, be2r, b2r = g2.reshape(1, D), be2.reshape(1, D), b2.reshape(1, D)

  xs0 = _tc_front(x, dinv, W0, g0r, be0r)
  p0 = spmm(xs0, src_p, dst_p, zeros_blk)
  h1, xs1 = _tc_mid(p0, xs0, dinv, b0r, W1, g1r, be1r)
  p1 = spmm(xs1, src_p, dst_p, zeros_blk)
  h2, xs2 = _tc_mid(p1, xs1, dinv, b1r, W2, g2r, be2r)
  p2 = spmm(xs2, src_p, dst_p, zeros_blk)
  return _tc_last(p2, xs2, dinv, b2r, h1, h2)


def kernel(x, edge_index, W0, b0, g0, be0, W1, b1, g1, be1, W2, b2, g2, be2):
  return _run(x, edge_index, W0, b0, g0, be0, W1, b1, g1, be1, W2, b2, g2,
              be2)


# speedup vs baseline: 26.7162x; 1.0147x over previous
import functools

import jax
import jax.numpy as jnp
from jax import lax
from jax.experimental import pallas as pl
from jax.experimental.pallas import tpu as pltpu
from jax.experimental.pallas import tpu_sc as plsc

N = 10000
D = 128
NC = 2
NS = 16
NW = NC * NS
CHUNK = 128
GRP = 8

ACC_ROWS = 10112
ACC_PER_TILE = ACC_ROWS // NS
EPS = 1e-5
PAD = -1

_MESH = plsc.VectorSubcoreMesh(core_axis_name="c", subcore_axis_name="s")


def _deg_body(nchunks, dst_hbm, zeros_hbm, ones_hbm, deg_out, idx_v, ones_v,
              stage_v, acc_sh):
  c = lax.axis_index("c")
  s = lax.axis_index("s")
  wid = c * NS + s
  pltpu.sync_copy(dst_hbm.at[wid], idx_v)
  pltpu.sync_copy(ones_hbm, ones_v)
  pltpu.sync_copy(zeros_hbm, stage_v)
  pltpu.sync_copy(stage_v, acc_sh.at[pl.ds(s * ACC_PER_TILE, ACC_PER_TILE)])
  plsc.subcore_barrier()
  for j in range(nchunks):
    pltpu.sync_copy(ones_v, acc_sh.at[plsc.Indices(idx_v.at[j], PAD)],
                    add=True)
  plsc.subcore_barrier()
  pltpu.sync_copy(acc_sh.at[pl.ds(s * ACC_PER_TILE, ACC_PER_TILE)], stage_v)
  pltpu.sync_copy(stage_v, deg_out.at[c, s, 0])


def _spmm_body(nchunks, xs_hbm, src_hbm, dst_hbm, zeros_hbm, pout_hbm,
               sidx_v, dstidx_v, rows_v, semi, semg0, semg1, acc_sh):
  c = lax.axis_index("c")
  s = lax.axis_index("s")
  wid = c * NS + s
  ngrp = nchunks // GRP
  pltpu.sync_copy(dst_hbm.at[wid], dstidx_v)
  pltpu.sync_copy(src_hbm.at[wid, pl.ds(0, GRP)], sidx_v.at[0])

  pltpu.sync_copy(zeros_hbm, rows_v.at[0])
  base = s * ACC_PER_TILE
  for k in range(ACC_PER_TILE // CHUNK):
    pltpu.sync_copy(rows_v.at[0], acc_sh.at[pl.ds(base + k * CHUNK, CHUNK)])
  rem = ACC_PER_TILE % CHUNK
  if rem:
    pltpu.sync_copy(
        rows_v.at[0, pl.ds(0, rem)],
        acc_sh.at[pl.ds(base + ACC_PER_TILE - rem, rem)])
  plsc.subcore_barrier()

  semg = (semg0, semg1)
  icp = None
  cp = pltpu.async_copy(xs_hbm.at[plsc.Indices(sidx_v.at[0, 0], PAD)],
                        rows_v.at[0], semg[0])
  for j in range(nchunks):
    g, i = divmod(j, GRP)
    if i == 0 and g + 1 < ngrp:
      icp = pltpu.async_copy(src_hbm.at[wid, pl.ds((g + 1) * GRP, GRP)],
                             sidx_v.at[(g + 1) % 2], semi)
    if j + 1 < nchunks:
      g1, i1 = divmod(j + 1, GRP)
      if i1 == 0:
        icp.wait()
      nxt = pltpu.async_copy(
          xs_hbm.at[plsc.Indices(sidx_v.at[g1 % 2, i1], PAD)],
          rows_v.at[(j + 1) % 2], semg[(j + 1) % 2])
    cp.wait()
    pltpu.sync_copy(rows_v.at[j % 2],
                    acc_sh.at[plsc.Indices(dstidx_v.at[j], PAD)], add=True)
    if j + 1 < nchunks:
      cp = nxt
  plsc.subcore_barrier()

  for k in range(ACC_PER_TILE // CHUNK):
    r0 = base + k * CHUNK
    pltpu.sync_copy(acc_sh.at[pl.ds(r0, CHUNK)], rows_v.at[0])
    pltpu.sync_copy(rows_v.at[0], pout_hbm.at[c, pl.ds(r0, CHUNK)])
  if rem:
    r0 = base + ACC_PER_TILE - rem
    pltpu.sync_copy(acc_sh.at[pl.ds(r0, rem)], rows_v.at[0, pl.ds(0, rem)])
    pltpu.sync_copy(rows_v.at[0, pl.ds(0, rem)],
                    pout_hbm.at[c, pl.ds(r0, rem)])


def _make_deg_kernel(nchunks):
  return pl.kernel(
      functools.partial(_deg_body, nchunks),
      out_type=jax.ShapeDtypeStruct((NC, NS, 1, ACC_PER_TILE), jnp.float32),
      mesh=_MESH,
      scratch_types=[
          pltpu.VMEM((nchunks, CHUNK), jnp.int32),
          pltpu.VMEM((CHUNK,), jnp.float32),
          pltpu.VMEM((ACC_PER_TILE,), jnp.float32),
          pltpu.VMEM_SHARED((ACC_ROWS,), jnp.float32),
      ],
  )


def _make_spmm_kernel(nchunks):
  return pl.kernel(
      functools.partial(_spmm_body, nchunks),
      out_type=jax.ShapeDtypeStruct((NC, ACC_ROWS, D), jnp.float32),
      mesh=_MESH,
      scratch_types=[
          pltpu.VMEM((2, GRP, CHUNK), jnp.int32),
          pltpu.VMEM((nchunks, CHUNK), jnp.int32),
          pltpu.VMEM((2, CHUNK, D), jnp.float32),
          pltpu.SemaphoreType.DMA,
          pltpu.SemaphoreType.DMA,
          pltpu.SemaphoreType.DMA,
          pltpu.VMEM_SHARED((ACC_ROWS, D), jnp.float32),
      ],
  )


def _bn_xw(h, g, be, w, dinv):
  mean = jnp.mean(h, axis=0, keepdims=True)
  dlt = h - mean
  var = jnp.mean(dlt * dlt, axis=0, keepdims=True)
  hb = g * dlt * lax.rsqrt(var + EPS) + be
  return dinv * jnp.dot(hb, w, preferred_element_type=jnp.float32)


def _tc_front_body(x_ref, deg_ref, w_ref, g_ref, be_ref, xs_ref, dinv_ref):
  dinv_row = lax.rsqrt(deg_ref[0:1, :] + deg_ref[1:2, :] + 1.0)
  dinv_ref[...] = dinv_row
  dinv = jnp.transpose(dinv_row[:, :N])
  xs_ref[...] = _bn_xw(x_ref[...], g_ref[...], be_ref[...], w_ref[...],
                       dinv)


def _tc_mid_body(p_ref, xs_ref, dinv_ref, b_ref, w_ref, g_ref, be_ref,
                 h_ref, xsn_ref):
  dinv = jnp.transpose(dinv_ref[:, :N])
  ssum = p_ref[0, :N] + p_ref[1, :N] + xs_ref[...]
  h = jnp.maximum(dinv * ssum + b_ref[...], 0.0)
  h_ref[...] = h
  xsn_ref[...] = _bn_xw(h, g_ref[...], be_ref[...], w_ref[...], dinv)


def _tc_last_body(p_ref, xs_ref, dinv_ref, b_ref, h1_ref, h2_ref, out_ref):
  dinv = jnp.transpose(dinv_ref[:, :N])
  ssum = p_ref[0, :N] + p_ref[1, :N] + xs_ref[...]
  h3 = jnp.maximum(dinv * ssum + b_ref[...], 0.0)
  out_ref[...] = jnp.concatenate([h1_ref[...], h2_ref[...], h3], axis=-1)


_tc_front = pl.pallas_call(
    _tc_front_body,
    out_shape=(
        jax.ShapeDtypeStruct((N, D), jnp.float32),
        jax.ShapeDtypeStruct((1, ACC_ROWS), jnp.float32),
    ),
)

_tc_mid = pl.pallas_call(
    _tc_mid_body,
    out_shape=(
        jax.ShapeDtypeStruct((N, D), jnp.float32),
        jax.ShapeDtypeStruct((N, D), jnp.float32),
    ),
)

_tc_last = pl.pallas_call(
    _tc_last_body,
    out_shape=jax.ShapeDtypeStruct((N, 3 * D), jnp.float32),
)


@jax.jit
def _run(x, edge_index, W0, b0, g0, be0, W1, b1, g1, be1, W2, b2, g2, be2):
  e = edge_index.shape[1]
  ncs = -(-e // (NW * CHUNK * GRP)) * GRP

  if e % NW == 0:
    ppt = ncs * CHUNK - e // NW
    padv = jnp.full((NW, ppt), PAD, jnp.int32)
    src_p = jnp.concatenate(
        [edge_index[0].reshape(NW, e // NW), padv], axis=1
    ).reshape(NW, ncs, CHUNK)
    dst_p = jnp.concatenate(
        [edge_index[1].reshape(NW, e // NW), padv], axis=1
    ).reshape(NW, ncs, CHUNK)
  else:
    pads = NW * ncs * CHUNK - e
    padv = jnp.full((pads,), PAD, jnp.int32)
    src_p = jnp.concatenate([edge_index[0], padv]).reshape(NW, ncs, CHUNK)
    dst_p = jnp.concatenate([edge_index[1], padv]).reshape(NW, ncs, CHUNK)

  zeros_deg = jnp.zeros((ACC_PER_TILE,), jnp.float32)
  zeros_blk = jnp.zeros((CHUNK, D), jnp.float32)
  ones_row = jnp.ones((CHUNK,), jnp.float32)

  deg_p = _make_deg_kernel(ncs)(dst_p, zeros_deg, ones_row)
  deg_p = deg_p.reshape(NC, ACC_ROWS)

  spmm = _make_spmm_kernel(ncs)
  g0r, be0r, b0r = g0.reshape(1, D), be0.reshape(1, D), b0.reshape(1, D)
  g1r, be1r, b1r = g1.reshape(1, D), be1.reshape(1, D), b1.reshape(1, D)
  g2r, be2r, b2r = g2.reshape(1, D), be2.reshape(1, D), b2.reshape(1, D)

  xs0, dinv = _tc_front(x, deg_p, W0, g0r, be0r)
  p0 = spmm(xs0, src_p, dst_p, zeros_blk)
  h1, xs1 = _tc_mid(p0, xs0, dinv, b0r, W1, g1r, be1r)
  p1 = spmm(xs1, src_p, dst_p, zeros_blk)
  h2, xs2 = _tc_mid(p1, xs1, dinv, b1r, W2, g2r, be2r)
  p2 = spmm(xs2, src_p, dst_p, zeros_blk)
  return _tc_last(p2, xs2, dinv, b2r, h1, h2)


def kernel(x, edge_index, W0, b0, g0, be0, W1, b1, g1, be1, W2, b2, g2, be2):
  return _run(x, edge_index, W0, b0, g0, be0, W1, b1, g1, be1, W2, b2, g2,
              be2)
